# baseline (speedup 1.0000x reference)
import functools

import jax
import jax.numpy as jnp
from jax import lax
from jax.experimental import pallas as pl
from jax.experimental.pallas import tpu as pltpu
from jax.experimental.pallas import tpu_sc as plsc

B = 64
NV = 256
P = 1024
ENC = 768
DEC = 384

NC = 2
NS = 16
NW = NC * NS
CH = 128
CPB = NV // CH
ROWS_PER_W = B // NW
NT = ROWS_PER_W * CPB


def _tc_body(x_ref, ids_ref, w_ref, bias_ref, apv_ref, amv_ref, pm_ref,
             mm_ref, pos_ref, xbv_ref, out_any, base_scr, fill_sem):
    bi = pl.program_id(0)

    @pl.when(bi == 0)
    def _():
        base_scr[:P] = pm_ref[...] + apv_ref[...] + pos_ref[:P]
        base_scr[P:] = mm_ref[...] + amv_ref[...] + pos_ref[P:]

    pltpu.make_async_copy(
        base_scr, out_any.at[pl.ds(bi * 2 * P, 2 * P)], fill_sem).start()

    @pl.when(bi >= 4)
    def _():
        pltpu.make_async_copy(
            base_scr, out_any.at[pl.ds(0, 2 * P)], fill_sem).wait()

    @pl.when(bi == B - 1)
    def _():
        for _ in range(4):
            pltpu.make_async_copy(
                base_scr, out_any.at[pl.ds(0, 2 * P)], fill_sem).wait()

    ids = ids_ref[0, 0]
    onehot = (ids[:, None] == lax.broadcasted_iota(jnp.int32, (NV, P), 1)
              ).astype(jnp.float32)
    posg = jnp.dot(onehot, pos_ref[:P], preferred_element_type=jnp.float32)
    xbv_ref[0] = (jnp.dot(x_ref[0], w_ref[...],
                          preferred_element_type=jnp.float32)
                  + bias_ref[...] + apv_ref[...] + posg)


def _tc_stage(x, ids3, w, bias, apv, amv, pm, mm, pos2):
    return pl.pallas_call(
        _tc_body,
        grid=(B,),
        in_specs=[
            pl.BlockSpec((1, NV, ENC), lambda i: (i, 0, 0)),
            pl.BlockSpec((1, 1, NV), lambda i: (i, 0, 0)),
            pl.BlockSpec((ENC, DEC), lambda i: (0, 0)),
            pl.BlockSpec((1, DEC), lambda i: (0, 0)),
            pl.BlockSpec((1, DEC), lambda i: (0, 0)),
            pl.BlockSpec((1, DEC), lambda i: (0, 0)),
            pl.BlockSpec((P, DEC), lambda i: (0, 0)),
            pl.BlockSpec((P, DEC), lambda i: (0, 0)),
            pl.BlockSpec((2 * P, DEC), lambda i: (0, 0)),
        ],
        out_shape=[
            jax.ShapeDtypeStruct((B, NV, DEC), jnp.float32),
            jax.ShapeDtypeStruct((B * 2 * P, DEC), jnp.float32),
        ],
        out_specs=[
            pl.BlockSpec((1, NV, DEC), lambda i: (i, 0, 0)),
            pl.BlockSpec(memory_space=pl.ANY),
        ],
        scratch_shapes=[pltpu.VMEM((2 * P, DEC), jnp.float32),
                        pltpu.SemaphoreType.DMA],
    )(x, ids3, w, bias, apv, amv, pm, mm, pos2)


def _sc_body(xbv_hbm, ids_hbm, out_hbm,
             idx_v, gidx2, rows2, gsem0, gsem1, ssem0, ssem1):
    cid = lax.axis_index("c")
    sid = lax.axis_index("s")
    wid = sid * NC + cid
    gsems = (gsem0, gsem1)
    ssems = (ssem0, ssem1)

    hgather = [None, None]
    hscatter = [None, None]

    def start_gather(t):
        buf = t % 2
        bb = wid * ROWS_PER_W + t // CPB
        c = t % CPB
        pltpu.sync_copy(ids_hbm.at[bb, pl.ds(c * CH, CH)], idx_v)
        off = bb * 2 * P
        for i in range(CH // 16):
            gidx2[buf, pl.ds(i * 16, 16)] = idx_v[pl.ds(i * 16, 16)] + off
        hgather[buf] = pltpu.async_copy(
            xbv_hbm.at[bb, pl.ds(c * CH, CH)], rows2.at[buf], gsems[buf])

    start_gather(0)
    for t in range(NT):
        buf = t % 2
        hgather[buf].wait()
        hscatter[buf] = pltpu.async_copy(
            rows2.at[buf], out_hbm.at[gidx2.at[buf]], ssems[buf])
        if t + 1 < NT:
            nbuf = (t + 1) % 2
            if hscatter[nbuf] is not None:
                hscatter[nbuf].wait()
            start_gather(t + 1)
    hscatter[(NT - 1) % 2].wait()


_sc_scatter = functools.partial(
    pl.kernel,
    out_type=(),
    mesh=plsc.VectorSubcoreMesh(core_axis_name="c", subcore_axis_name="s"),
    scratch_types=[
        pltpu.VMEM((CH,), jnp.int32),
        pltpu.VMEM((2, CH), jnp.int32),
        pltpu.VMEM((2, CH, DEC), jnp.float32),
        pltpu.SemaphoreType.DMA,
        pltpu.SemaphoreType.DMA,
        pltpu.SemaphoreType.DMA,
        pltpu.SemaphoreType.DMA,
    ],
)(_sc_body)


def kernel(x, visible_ids, partial_view_id, W, b, partial_mask, masked_mask,
           pos, view1, view2):
    cond = partial_view_id == 0
    apv = jnp.where(cond, view1, view2).reshape(1, DEC)
    amv = jnp.where(cond, view2, view1).reshape(1, DEC)
    ids3 = visible_ids.reshape(B, 1, NV)
    xbv, out0 = _tc_stage(x, ids3, W, b.reshape(1, DEC), apv, amv,
                          partial_mask.reshape(P, DEC),
                          masked_mask.reshape(P, DEC),
                          pos.reshape(2 * P, DEC))
    o_ref = jax.new_ref(out0)
    _sc_scatter(xbv, visible_ids, o_ref)
    return jax.freeze(o_ref).reshape(B, 2 * P, DEC)


# speedup vs baseline: 1.0038x; 1.0038x over previous
import functools

import jax
import jax.numpy as jnp
from jax import lax
from jax.experimental import pallas as pl
from jax.experimental.pallas import tpu as pltpu
from jax.experimental.pallas import tpu_sc as plsc

B = 64
NV = 256
P = 1024
ENC = 768
DEC = 384

NC = 2
NS = 16
NW = NC * NS
CH = 128
CPB = NV // CH
ROWS_PER_W = B // NW
NT = ROWS_PER_W * CPB


def _tc_body(x_ref, ids_ref, w_ref, bias_ref, apv_ref, amv_ref, pm_ref,
             mm_ref, pos_ref, xbv_ref, out_any, base_scr, fill_sem0,
             fill_sem1):
    bi = pl.program_id(0)

    @pl.when(bi == 0)
    def _():
        base_scr[:P] = pm_ref[...] + apv_ref[...] + pos_ref[:P]
        base_scr[P:] = mm_ref[...] + amv_ref[...] + pos_ref[P:]

    pltpu.make_async_copy(
        base_scr.at[pl.ds(0, P)],
        out_any.at[pl.ds(bi * 2 * P, P)], fill_sem0).start()
    pltpu.make_async_copy(
        base_scr.at[pl.ds(P, P)],
        out_any.at[pl.ds(bi * 2 * P + P, P)], fill_sem1).start()

    @pl.when(bi >= 2)
    def _():
        pltpu.make_async_copy(
            base_scr.at[pl.ds(0, P)], out_any.at[pl.ds(0, P)],
            fill_sem0).wait()
        pltpu.make_async_copy(
            base_scr.at[pl.ds(0, P)], out_any.at[pl.ds(0, P)],
            fill_sem1).wait()

    @pl.when(bi == B - 1)
    def _():
        for _ in range(2):
            pltpu.make_async_copy(
                base_scr.at[pl.ds(0, P)], out_any.at[pl.ds(0, P)],
                fill_sem0).wait()
            pltpu.make_async_copy(
                base_scr.at[pl.ds(0, P)], out_any.at[pl.ds(0, P)],
                fill_sem1).wait()

    ids = ids_ref[0, 0]
    onehot = (ids[:, None] == lax.broadcasted_iota(jnp.int32, (NV, P), 1)
              ).astype(jnp.float32)
    posg = jnp.dot(onehot, pos_ref[:P], preferred_element_type=jnp.float32)
    xbv_ref[0] = (jnp.dot(x_ref[0], w_ref[...],
                          preferred_element_type=jnp.float32)
                  + bias_ref[...] + apv_ref[...] + posg)


def _tc_stage(x, ids3, w, bias, apv, amv, pm, mm, pos2):
    return pl.pallas_call(
        _tc_body,
        grid=(B,),
        in_specs=[
            pl.BlockSpec((1, NV, ENC), lambda i: (i, 0, 0)),
            pl.BlockSpec((1, 1, NV), lambda i: (i, 0, 0)),
            pl.BlockSpec((ENC, DEC), lambda i: (0, 0)),
            pl.BlockSpec((1, DEC), lambda i: (0, 0)),
            pl.BlockSpec((1, DEC), lambda i: (0, 0)),
            pl.BlockSpec((1, DEC), lambda i: (0, 0)),
            pl.BlockSpec((P, DEC), lambda i: (0, 0)),
            pl.BlockSpec((P, DEC), lambda i: (0, 0)),
            pl.BlockSpec((2 * P, DEC), lambda i: (0, 0)),
        ],
        out_shape=[
            jax.ShapeDtypeStruct((B, NV, DEC), jnp.float32),
            jax.ShapeDtypeStruct((B * 2 * P, DEC), jnp.float32),
        ],
        out_specs=[
            pl.BlockSpec((1, NV, DEC), lambda i: (i, 0, 0)),
            pl.BlockSpec(memory_space=pl.ANY),
        ],
        scratch_shapes=[pltpu.VMEM((2 * P, DEC), jnp.float32),
                        pltpu.SemaphoreType.DMA,
                        pltpu.SemaphoreType.DMA],
    )(x, ids3, w, bias, apv, amv, pm, mm, pos2)


def _sc_body(xbv_hbm, ids_hbm, out_hbm,
             idx_v, gidx2, rows2, gsem0, gsem1, ssem0, ssem1):
    cid = lax.axis_index("c")
    sid = lax.axis_index("s")
    wid = sid * NC + cid
    gsems = (gsem0, gsem1)
    ssems = (ssem0, ssem1)

    hgather = [None, None]
    hscatter = [None, None]

    def start_gather(t):
        buf = t % 2
        bb = wid * ROWS_PER_W + t // CPB
        c = t % CPB
        pltpu.sync_copy(ids_hbm.at[bb, pl.ds(c * CH, CH)], idx_v)
        off = bb * 2 * P
        for i in range(CH // 16):
            gidx2[buf, pl.ds(i * 16, 16)] = idx_v[pl.ds(i * 16, 16)] + off
        hgather[buf] = pltpu.async_copy(
            xbv_hbm.at[bb, pl.ds(c * CH, CH)], rows2.at[buf], gsems[buf])

    start_gather(0)
    for t in range(NT):
        buf = t % 2
        hgather[buf].wait()
        hscatter[buf] = pltpu.async_copy(
            rows2.at[buf], out_hbm.at[gidx2.at[buf]], ssems[buf])
        if t + 1 < NT:
            nbuf = (t + 1) % 2
            if hscatter[nbuf] is not None:
                hscatter[nbuf].wait()
            start_gather(t + 1)
    hscatter[(NT - 1) % 2].wait()


_sc_scatter = functools.partial(
    pl.kernel,
    out_type=(),
    mesh=plsc.VectorSubcoreMesh(core_axis_name="c", subcore_axis_name="s"),
    scratch_types=[
        pltpu.VMEM((CH,), jnp.int32),
        pltpu.VMEM((2, CH), jnp.int32),
        pltpu.VMEM((2, CH, DEC), jnp.float32),
        pltpu.SemaphoreType.DMA,
        pltpu.SemaphoreType.DMA,
        pltpu.SemaphoreType.DMA,
        pltpu.SemaphoreType.DMA,
    ],
)(_sc_body)


def kernel(x, visible_ids, partial_view_id, W, b, partial_mask, masked_mask,
           pos, view1, view2):
    cond = partial_view_id == 0
    apv = jnp.where(cond, view1, view2).reshape(1, DEC)
    amv = jnp.where(cond, view2, view1).reshape(1, DEC)
    ids3 = visible_ids.reshape(B, 1, NV)
    xbv, out0 = _tc_stage(x, ids3, W, b.reshape(1, DEC), apv, amv,
                          partial_mask.reshape(P, DEC),
                          masked_mask.reshape(P, DEC),
                          pos.reshape(2 * P, DEC))
    o_ref = jax.new_ref(out0)
    _sc_scatter(xbv, visible_ids, o_ref)
    return jax.freeze(o_ref).reshape(B, 2 * P, DEC)
